# baseline (speedup 1.0000x reference)
import functools

import jax
import jax.numpy as jnp
from jax import lax
from jax.experimental import pallas as pl
from jax.experimental.pallas import tpu as pltpu

NUM_EMBEDDINGS = 1000000
EMBED_DIM = 64
BATCH = 16384
NSEM = 8
GROUPS = BATCH // NSEM


def _body(idx_s, w_hbm, out_hbm, buf, sems):
    def issue(o, _):
        for j in range(NSEM):
            i = o * NSEM + j
            r = idx_s[i]
            pltpu.make_async_copy(
                w_hbm.at[pl.ds(r, 1)],
                buf.at[pl.ds(i, 1)],
                sems.at[j],
            ).start()
        return 0

    lax.fori_loop(0, GROUPS, issue, 0, unroll=4)
    for j in range(NSEM):
        pltpu.make_async_copy(
            w_hbm.at[pl.ds(0, GROUPS)],
            buf.at[pl.ds(j * GROUPS, GROUPS)],
            sems.at[j],
        ).wait()
    pltpu.sync_copy(buf, out_hbm)


_gather_tc = pl.pallas_call(
    _body,
    out_shape=jax.ShapeDtypeStruct((BATCH, EMBED_DIM), jnp.float32),
    in_specs=[
        pl.BlockSpec(memory_space=pltpu.SMEM),
        pl.BlockSpec(memory_space=pl.ANY),
    ],
    out_specs=pl.BlockSpec(memory_space=pl.ANY),
    scratch_shapes=[
        pltpu.VMEM((BATCH, EMBED_DIM), jnp.float32),
        pltpu.SemaphoreType.DMA((NSEM,)),
    ],
)


def kernel(batch, w):
    return _gather_tc(batch.astype(jnp.int32), w)


# speedup vs baseline: 1.0339x; 1.0339x over previous
import functools

import jax
import jax.numpy as jnp
from jax import lax
from jax.experimental import pallas as pl
from jax.experimental.pallas import tpu as pltpu
from jax.experimental.pallas import tpu_sc as plsc

NUM_EMBEDDINGS = 1000000
EMBED_DIM = 64
BATCH = 16384
NUM_CORES = 2
NUM_SUBCORES = 16
NUM_WORKERS = NUM_CORES * NUM_SUBCORES
LANES = 16

SPLIT = 8192

_mesh = plsc.VectorSubcoreMesh(core_axis_name="c", subcore_axis_name="s")


def _make_sc(n_rows):
    b_per_w = n_rows // NUM_WORKERS

    @functools.partial(
        pl.kernel,
        mesh=_mesh,
        out_type=jax.ShapeDtypeStruct((n_rows, EMBED_DIM), jnp.float32),
        scratch_types=[
            pltpu.VMEM((b_per_w,), jnp.int32),
            pltpu.VMEM((b_per_w, EMBED_DIM), jnp.float32),
            pltpu.SemaphoreType.DMA,
        ],
    )
    def gather_sc(idx_hbm, table_hbm, out_hbm, idx_v, rows_v, sem):
        wid = lax.axis_index("s") * NUM_CORES + lax.axis_index("c")
        base = wid * b_per_w
        pltpu.sync_copy(idx_hbm.at[pl.ds(base, b_per_w)], idx_v)

        @pl.loop(0, b_per_w // LANES)
        def _group(g):
            vec = idx_v[pl.ds(g * LANES, LANES)]
            for j in range(LANES):
                r = vec[j]
                i = g * LANES + j
                pltpu.make_async_copy(
                    table_hbm.at[pl.ds(r, 1)],
                    rows_v.at[pl.ds(i, 1)],
                    sem,
                ).start()

        pltpu.make_async_copy(
            table_hbm.at[pl.ds(0, b_per_w)],
            rows_v,
            sem,
        ).wait()
        pltpu.sync_copy(rows_v, out_hbm.at[pl.ds(base, b_per_w)])

    return gather_sc


def _make_tc(n_rows, nsem=8, unroll=8):
    groups = n_rows // nsem

    def body(idx_s, w_hbm, out_hbm, buf, sems):
        def issue(o, _):
            for j in range(nsem):
                i = o * nsem + j
                r = idx_s[i]
                pltpu.make_async_copy(
                    w_hbm.at[pl.ds(r, 1)],
                    buf.at[pl.ds(i, 1)],
                    sems.at[j],
                ).start()
            return 0

        lax.fori_loop(0, groups, issue, 0, unroll=unroll)
        for j in range(nsem):
            pltpu.make_async_copy(
                w_hbm.at[pl.ds(0, groups)],
                buf.at[pl.ds(j * groups, groups)],
                sems.at[j],
            ).wait()
        pltpu.sync_copy(buf, out_hbm)

    return pl.pallas_call(
        body,
        out_shape=jax.ShapeDtypeStruct((n_rows, EMBED_DIM), jnp.float32),
        in_specs=[
            pl.BlockSpec(memory_space=pltpu.SMEM),
            pl.BlockSpec(memory_space=pl.ANY),
        ],
        out_specs=pl.BlockSpec(memory_space=pl.ANY),
        scratch_shapes=[
            pltpu.VMEM((n_rows, EMBED_DIM), jnp.float32),
            pltpu.SemaphoreType.DMA((nsem,)),
        ],
    )


_gather_sc = _make_sc(SPLIT)
_gather_tc = _make_tc(BATCH - SPLIT)


def kernel(batch, w):
    idx = batch.astype(jnp.int32)
    out_sc = _gather_sc(idx[:SPLIT], w)
    out_tc = _gather_tc(idx[SPLIT:], w)
    return jnp.concatenate([out_sc, out_tc], axis=0)
